# baseline (speedup 1.0000x reference)
import functools

import jax
import jax.numpy as jnp
from jax.experimental import pallas as pl
from jax.experimental.pallas import tpu as pltpu


def _encode_body(x_ref, w_ref, be_ref, bd_ref, out_ref, cmax_ref):
    xc = x_ref[...] - bd_ref[...][None, :]
    acc = jax.lax.dot_general(
        xc, w_ref[...], (((1,), (1,)), ((), ())),
        preferred_element_type=jnp.float32)
    post = jnp.maximum(acc + be_ref[...][None, :], 0.0)
    out_ref[...] = post
    bt, ft = post.shape
    cmax_ref[...] = jnp.max(post.reshape(bt, ft // 128, 128), axis=1)


def _thresh_body(post_ref, cmax_ref, thr_ref, lo_ref, hi_ref, *, k):
    post = post_ref[...]
    bt = post.shape[0]
    bits = jax.lax.bitcast_convert_type(post, jnp.int32)
    cmax = cmax_ref[...]
    nc = cmax.shape[1]
    c2 = jnp.max(cmax.reshape(bt, nc // 128, 128), axis=1)
    if 128 >= k:
        lo0 = jnp.min(c2, axis=1, keepdims=True)
    else:
        lo0 = jnp.zeros((bt, 1), jnp.float32)
    hi0 = jnp.max(c2, axis=1, keepdims=True)
    lo_ref[...] = jax.lax.bitcast_convert_type(lo0, jnp.int32)
    hi_ref[...] = jax.lax.bitcast_convert_type(hi0, jnp.int32)

    def step(lo, hi):
        mid = lo + jax.lax.shift_right_logical(hi - lo + 1, 1)
        cnt = jnp.sum((bits >= mid).astype(jnp.int32), axis=1, keepdims=True)
        ge = cnt >= k
        new_lo = jnp.where(ge, mid, lo)
        new_hi = jnp.where(ge, hi, mid - 1)
        new_hi = jnp.where(cnt == k, new_lo, new_hi)
        return new_lo, new_hi

    def cond(n_active):
        return n_active > 0

    def body(n_active):
        lo, hi = step(lo_ref[...], hi_ref[...])
        lo, hi = step(lo, hi)
        lo_ref[...] = lo
        hi_ref[...] = hi
        return jnp.sum((lo < hi).astype(jnp.int32))

    jax.lax.while_loop(cond, body, jnp.int32(bt))
    thr_ref[...] = lo_ref[...]


def _decode_body(post_ref, thr_ref, w_ref, bd_ref, out_ref):
    j = pl.program_id(1)
    post = post_ref[...]
    bits = jax.lax.bitcast_convert_type(post, jnp.int32)
    feats = jnp.where(bits >= thr_ref[...], post, 0.0)
    part = jax.lax.dot_general(
        feats.astype(jnp.bfloat16), w_ref[...].astype(jnp.bfloat16),
        (((1,), (1,)), ((), ())),
        preferred_element_type=jnp.float32)

    @pl.when(j == 0)
    def _():
        out_ref[...] = part + bd_ref[...][None, :]

    @pl.when(j != 0)
    def _():
        out_ref[...] += part


def _forward(x, W_enc, b_enc, W_dec, b_dec, k, stage=3):
    B, D = x.shape
    F = W_enc.shape[0]
    Bt = min(512, B)
    Ft = min(1024, F)
    Bt2 = min(256, B)
    nb, nf, nb2 = B // Bt, F // Ft, B // Bt2

    post = pl.pallas_call(
        _encode_body,
        grid=(nf, nb),
        in_specs=[
            pl.BlockSpec((Bt, D), lambda j, i: (i, 0)),
            pl.BlockSpec((Ft, D), lambda j, i: (j, 0)),
            pl.BlockSpec((Ft,), lambda j, i: (j,)),
            pl.BlockSpec((D,), lambda j, i: (0,)),
        ],
        out_specs=(
            pl.BlockSpec((Bt, Ft), lambda j, i: (i, j)),
            pl.BlockSpec((Bt, 128), lambda j, i: (i, j)),
        ),
        out_shape=(
            jax.ShapeDtypeStruct((B, F), jnp.float32),
            jax.ShapeDtypeStruct((B, nf * 128), jnp.float32),
        ),
    )(x, W_enc, b_enc, b_dec)
    post, cmax = post
    if stage == 1:
        return post

    thr = pl.pallas_call(
        functools.partial(_thresh_body, k=k),
        grid=(nb2,),
        in_specs=[
            pl.BlockSpec((Bt2, F), lambda i: (i, 0)),
            pl.BlockSpec((Bt2, nf * 128), lambda i: (i, 0)),
        ],
        out_specs=pl.BlockSpec((Bt2, 1), lambda i: (i, 0)),
        out_shape=jax.ShapeDtypeStruct((B, 1), jnp.int32),
        scratch_shapes=[
            pltpu.VMEM((Bt2, 1), jnp.int32),
            pltpu.VMEM((Bt2, 1), jnp.int32),
        ],
    )(post, cmax)
    if stage == 2:
        return thr

    Btd = min(1024, B)
    Ftd = min(1024, F)
    nbd, nfd = B // Btd, F // Ftd
    recon = pl.pallas_call(
        _decode_body,
        grid=(nbd, nfd),
        in_specs=[
            pl.BlockSpec((Btd, Ftd), lambda i, j: (i, j)),
            pl.BlockSpec((Btd, 1), lambda i, j: (i, 0)),
            pl.BlockSpec((D, Ftd), lambda i, j: (0, j)),
            pl.BlockSpec((D,), lambda i, j: (0,)),
        ],
        out_specs=pl.BlockSpec((Btd, D), lambda i, j: (i, 0)),
        out_shape=jax.ShapeDtypeStruct((B, D), jnp.float32),
    )(post, thr, W_dec, b_dec)
    return recon


def kernel(x, W_enc, b_enc, W_dec, b_dec):
    return _forward(x, W_enc, b_enc, W_dec, b_dec, k=64, stage=3)


# speedup vs baseline: 1.0719x; 1.0719x over previous
import functools

import jax
import jax.numpy as jnp
from jax.experimental import pallas as pl
from jax.experimental.pallas import tpu as pltpu


def _encode_body(x_ref, w_ref, be_ref, bd_ref, out_ref):
    xc = x_ref[...] - bd_ref[...][None, :]
    acc = jax.lax.dot_general(
        xc, w_ref[...], (((1,), (1,)), ((), ())),
        preferred_element_type=jnp.float32)
    out_ref[...] = jnp.maximum(acc + be_ref[...][None, :], 0.0)


def _thresh_body(post_ref, thr_ref, lo_ref, hi_ref, *, k):
    post = post_ref[...]
    bt, f = post.shape
    bits = jax.lax.bitcast_convert_type(post, jnp.int32)
    cmax = jnp.max(post.reshape(bt, k, f // k), axis=2)
    lo_ref[...] = jax.lax.bitcast_convert_type(
        jnp.min(cmax, axis=1, keepdims=True), jnp.int32)
    hi_ref[...] = jax.lax.bitcast_convert_type(
        jnp.max(cmax, axis=1, keepdims=True), jnp.int32)

    def step(lo, hi):
        mid = lo + jax.lax.shift_right_logical(hi - lo + 1, 1)
        cnt = jnp.sum((bits >= mid).astype(jnp.int32), axis=1, keepdims=True)
        ge = cnt >= k
        new_lo = jnp.where(ge, mid, lo)
        new_hi = jnp.where(ge, hi, mid - 1)
        new_hi = jnp.where(cnt == k, new_lo, new_hi)
        return new_lo, new_hi

    def cond(n_active):
        return n_active > 0

    def body(n_active):
        lo, hi = step(lo_ref[...], hi_ref[...])
        lo, hi = step(lo, hi)
        lo_ref[...] = lo
        hi_ref[...] = hi
        return jnp.sum((lo < hi).astype(jnp.int32))

    jax.lax.while_loop(cond, body, jnp.int32(bt))
    thr_ref[...] = lo_ref[...]


def _decode_body(post_ref, thr_ref, w_ref, bd_ref, out_ref):
    j = pl.program_id(1)
    post = post_ref[...]
    bits = jax.lax.bitcast_convert_type(post, jnp.int32)
    feats = jnp.where(bits >= thr_ref[...], post, 0.0)
    part = jax.lax.dot_general(
        feats.astype(jnp.bfloat16), w_ref[...].astype(jnp.bfloat16),
        (((1,), (1,)), ((), ())),
        preferred_element_type=jnp.float32)

    @pl.when(j == 0)
    def _():
        out_ref[...] = part + bd_ref[...][None, :]

    @pl.when(j != 0)
    def _():
        out_ref[...] += part


def _forward(x, W_enc, b_enc, W_dec, b_dec, k, stage=3):
    B, D = x.shape
    F = W_enc.shape[0]
    Bt = min(512, B)
    Ft = min(2048, F)
    Bt2 = min(256, B)
    nb, nf, nb2 = B // Bt, F // Ft, B // Bt2

    post = pl.pallas_call(
        _encode_body,
        grid=(nf, nb),
        in_specs=[
            pl.BlockSpec((Bt, D), lambda j, i: (i, 0)),
            pl.BlockSpec((Ft, D), lambda j, i: (j, 0)),
            pl.BlockSpec((Ft,), lambda j, i: (j,)),
            pl.BlockSpec((D,), lambda j, i: (0,)),
        ],
        out_specs=pl.BlockSpec((Bt, Ft), lambda j, i: (i, j)),
        out_shape=jax.ShapeDtypeStruct((B, F), jnp.float32),
    )(x, W_enc, b_enc, b_dec)
    if stage == 1:
        return post

    thr = pl.pallas_call(
        functools.partial(_thresh_body, k=k),
        grid=(nb2,),
        in_specs=[pl.BlockSpec((Bt2, F), lambda i: (i, 0))],
        out_specs=pl.BlockSpec((Bt2, 1), lambda i: (i, 0)),
        out_shape=jax.ShapeDtypeStruct((B, 1), jnp.int32),
        scratch_shapes=[
            pltpu.VMEM((Bt2, 1), jnp.int32),
            pltpu.VMEM((Bt2, 1), jnp.int32),
        ],
    )(post)
    if stage == 2:
        return thr

    Btd = min(1024, B)
    Ftd = min(1024, F)
    nbd, nfd = B // Btd, F // Ftd
    recon = pl.pallas_call(
        _decode_body,
        grid=(nbd, nfd),
        in_specs=[
            pl.BlockSpec((Btd, Ftd), lambda i, j: (i, j)),
            pl.BlockSpec((Btd, 1), lambda i, j: (i, 0)),
            pl.BlockSpec((D, Ftd), lambda i, j: (0, j)),
            pl.BlockSpec((D,), lambda i, j: (0,)),
        ],
        out_specs=pl.BlockSpec((Btd, D), lambda i, j: (i, 0)),
        out_shape=jax.ShapeDtypeStruct((B, D), jnp.float32),
    )(post, thr, W_dec, b_dec)
    return recon


def kernel(x, W_enc, b_enc, W_dec, b_dec):
    return _forward(x, W_enc, b_enc, W_dec, b_dec, k=64, stage=3)
